# baseline (speedup 1.0000x reference)
import functools

import jax
import jax.numpy as jnp
from jax import lax
from jax.experimental import pallas as pl
from jax.experimental.pallas import tpu as pltpu
from jax.experimental.pallas import tpu_sc as plsc

N = 10000
D = 128
E = 320000

NC = 2
NS = 16
NW = NC * NS
CHUNK = 128
NB = 1
NH = 2
PER_W = -(-E // (NW * CHUNK * NB * NH)) * CHUNK * NB * NH
K = PER_W // CHUNK
K2 = K // NH
E_PAD = PER_W * NW
STRIPE = 632
STRIPE_LAST = N - STRIPE * (NS - 1)
N_PAD = N + 8

ROW_BLK = 1000


def _sc_aggregate(x, rows_r, cols_r):
    mesh = plsc.VectorSubcoreMesh(
        core_axis_name="c", subcore_axis_name="s",
        num_cores=NC, num_subcores=NS)

    @functools.partial(
        pl.kernel,
        out_type=jax.ShapeDtypeStruct((NC, N, D), jnp.float32),
        mesh=mesh,
        scratch_types=[
            pltpu.VMEM((K2, CHUNK), jnp.int32),
            pltpu.VMEM((K2, CHUNK), jnp.int32),
            [pltpu.VMEM((CHUNK, D), jnp.float32) for _ in range(NB)],
            pltpu.VMEM_SHARED((N_PAD, D), jnp.float32),
            [pltpu.SemaphoreType.DMA for _ in range(NB)],
        ],
    )
    def agg_kernel(x_hbm, rows_hbm, cols_hbm, out_hbm,
                   row_v, col_v, bufs, agg_sh, sems):
        c = lax.axis_index("c")
        s = lax.axis_index("s")
        off = pl.multiple_of(s * STRIPE, 8)

        @pl.when(s < NS - 1)
        def _():
            pltpu.sync_copy(x_hbm.at[pl.ds(off, STRIPE)],
                            agg_sh.at[pl.ds(off, STRIPE)])

        @pl.when(s == NS - 1)
        def _():
            pltpu.sync_copy(x_hbm.at[pl.ds((NS - 1) * STRIPE, STRIPE_LAST)],
                            agg_sh.at[pl.ds((NS - 1) * STRIPE, STRIPE_LAST)])

        plsc.subcore_barrier()

        for h in range(NH):
            pltpu.sync_copy(rows_hbm.at[c, s, h], row_v)
            pltpu.sync_copy(cols_hbm.at[c, s, h], col_v)

            @pl.loop(0, K2)
            def _(j):
                pltpu.async_copy(x_hbm.at[col_v.at[j]], bufs[0],
                                 sems[0]).wait()
                pltpu.sync_copy(bufs[0], agg_sh.at[row_v.at[j]], add=True)

        plsc.subcore_barrier()

        @pl.when(s < NS - 1)
        def _():
            pltpu.sync_copy(agg_sh.at[pl.ds(off, STRIPE)],
                            out_hbm.at[c, pl.ds(off, STRIPE)])

        @pl.when(s == NS - 1)
        def _():
            pltpu.sync_copy(
                agg_sh.at[pl.ds((NS - 1) * STRIPE, STRIPE_LAST)],
                out_hbm.at[c, pl.ds((NS - 1) * STRIPE, STRIPE_LAST)])

    return agg_kernel(x, rows_r, cols_r)


def _mlp_block(x_ref, a0_ref, a1_ref, w1_ref, b1_ref, w2_ref, b2_ref, o_ref):
    a = a0_ref[0] + a1_ref[0] - x_ref[...]
    h = jnp.dot(a, w1_ref[...], preferred_element_type=jnp.float32,
                precision=lax.Precision.HIGHEST) + b1_ref[...]
    h = jnp.maximum(h, 0.0)
    o_ref[...] = jnp.dot(h, w2_ref[...], preferred_element_type=jnp.float32,
                         precision=lax.Precision.HIGHEST) + b2_ref[...]


def _tc_mlp(x, agg, W1, b1, W2, b2):
    nb = N // ROW_BLK
    return pl.pallas_call(
        _mlp_block,
        grid=(nb,),
        in_specs=[
            pl.BlockSpec((ROW_BLK, D), lambda i: (i, 0)),
            pl.BlockSpec((1, ROW_BLK, D), lambda i: (0, i, 0)),
            pl.BlockSpec((1, ROW_BLK, D), lambda i: (1, i, 0)),
            pl.BlockSpec((D, D), lambda i: (0, 0)),
            pl.BlockSpec((1, D), lambda i: (0, 0)),
            pl.BlockSpec((D, D), lambda i: (0, 0)),
            pl.BlockSpec((1, D), lambda i: (0, 0)),
        ],
        out_specs=pl.BlockSpec((ROW_BLK, D), lambda i: (i, 0)),
        out_shape=jax.ShapeDtypeStruct((N, D), jnp.float32),
    )(x, agg, agg, W1, b1.reshape(1, D), W2, b2.reshape(1, D))


def kernel(x, edge_index, W1, b1, W2, b2):
    rows = edge_index[:, 0]
    cols = edge_index[:, 1]
    pad = E_PAD - E
    rows_p = jnp.concatenate(
        [rows, jnp.full((pad,), N, dtype=jnp.int32)]
    ).reshape(NC, NS, NH, K2, CHUNK)
    cols_p = jnp.concatenate(
        [cols, jnp.zeros((pad,), dtype=jnp.int32)]
    ).reshape(NC, NS, NH, K2, CHUNK)
    agg = _sc_aggregate(x, rows_p, cols_p)
    return _tc_mlp(x, agg, W1, b1, W2, b2)


# speedup vs baseline: 1.1164x; 1.1164x over previous
import functools

import jax
import jax.numpy as jnp
from jax import lax
from jax.experimental import pallas as pl
from jax.experimental.pallas import tpu as pltpu
from jax.experimental.pallas import tpu_sc as plsc

N = 10000
D = 128
E = 320000

NC = 2
NS = 16
NW = NC * NS
CHUNK = 128
NB = 2
PER_W = -(-E // (NW * CHUNK * NB)) * CHUNK * NB
K = PER_W // CHUNK
E_PAD = PER_W * NW
STRIPE = 632
STRIPE_LAST = N - STRIPE * (NS - 1)
N_PAD = N + 8

ROW_BLK = 1000


def _sc_aggregate(x, rows_r, cols_r):
    mesh = plsc.VectorSubcoreMesh(
        core_axis_name="c", subcore_axis_name="s",
        num_cores=NC, num_subcores=NS)

    @functools.partial(
        pl.kernel,
        out_type=jax.ShapeDtypeStruct((NC, N, D), jnp.float32),
        mesh=mesh,
        scratch_types=[
            pltpu.VMEM((K, CHUNK), jnp.int32),
            pltpu.VMEM((1, CHUNK), jnp.int32),
            pltpu.VMEM((1, CHUNK), jnp.int32),
            pltpu.VMEM((CHUNK, D), jnp.float32),
            pltpu.VMEM((CHUNK, D), jnp.float32),
            pltpu.VMEM_SHARED((N_PAD, D), jnp.float32),
            pltpu.SemaphoreType.DMA,
            pltpu.SemaphoreType.DMA,
            pltpu.SemaphoreType.DMA,
            pltpu.SemaphoreType.DMA,
        ],
    )
    def agg_kernel(x_hbm, rows_hbm, cols_hbm, out_hbm,
                   col_v, rowb0, rowb1, buf0, buf1, agg_sh,
                   semg0, semg1, semr0, semr1):
        c = lax.axis_index("c")
        s = lax.axis_index("s")
        pltpu.sync_copy(cols_hbm.at[c, s], col_v)
        off = pl.multiple_of(s * STRIPE, 8)

        @pl.when(s < NS - 1)
        def _():
            pltpu.sync_copy(x_hbm.at[pl.ds(off, STRIPE)],
                            agg_sh.at[pl.ds(off, STRIPE)])

        @pl.when(s == NS - 1)
        def _():
            pltpu.sync_copy(x_hbm.at[pl.ds((NS - 1) * STRIPE, STRIPE_LAST)],
                            agg_sh.at[pl.ds((NS - 1) * STRIPE, STRIPE_LAST)])

        plsc.subcore_barrier()

        pltpu.async_copy(rows_hbm.at[c, s, pl.ds(0, 1)], rowb0, semr0)
        pltpu.async_copy(rows_hbm.at[c, s, pl.ds(1, 1)], rowb1, semr1)
        pltpu.async_copy(x_hbm.at[col_v.at[0]], buf0, semg0)

        @pl.loop(0, K // NB)
        def _(g):
            j = g * NB
            pltpu.async_copy(x_hbm.at[col_v.at[j + 1]], buf1, semg1)
            pltpu.make_async_copy(x_hbm.at[col_v.at[j]], buf0, semg0).wait()
            pltpu.make_async_copy(
                rows_hbm.at[c, s, pl.ds(j, 1)], rowb0, semr0).wait()
            pltpu.sync_copy(buf0, agg_sh.at[rowb0.at[0]], add=True)

            @pl.when(j + 2 < K)
            def _():
                pltpu.async_copy(
                    rows_hbm.at[c, s, pl.ds(j + 2, 1)], rowb0, semr0)
                pltpu.async_copy(x_hbm.at[col_v.at[j + 2]], buf0, semg0)

            pltpu.make_async_copy(x_hbm.at[col_v.at[j + 1]], buf1,
                                  semg1).wait()
            pltpu.make_async_copy(
                rows_hbm.at[c, s, pl.ds(j + 1, 1)], rowb1, semr1).wait()
            pltpu.sync_copy(buf1, agg_sh.at[rowb1.at[0]], add=True)

            @pl.when(j + 3 < K)
            def _():
                pltpu.async_copy(
                    rows_hbm.at[c, s, pl.ds(j + 3, 1)], rowb1, semr1)

        plsc.subcore_barrier()

        @pl.when(s < NS - 1)
        def _():
            pltpu.sync_copy(agg_sh.at[pl.ds(off, STRIPE)],
                            out_hbm.at[c, pl.ds(off, STRIPE)])

        @pl.when(s == NS - 1)
        def _():
            pltpu.sync_copy(
                agg_sh.at[pl.ds((NS - 1) * STRIPE, STRIPE_LAST)],
                out_hbm.at[c, pl.ds((NS - 1) * STRIPE, STRIPE_LAST)])

    return agg_kernel(x, rows_r, cols_r)


def _mlp_block(x_ref, a0_ref, a1_ref, w1_ref, b1_ref, w2_ref, b2_ref, o_ref):
    a = a0_ref[0] + a1_ref[0] - x_ref[...]
    h = jnp.dot(a, w1_ref[...], preferred_element_type=jnp.float32,
                precision=lax.Precision.HIGHEST) + b1_ref[...]
    h = jnp.maximum(h, 0.0)
    o_ref[...] = jnp.dot(h, w2_ref[...], preferred_element_type=jnp.float32,
                         precision=lax.Precision.HIGHEST) + b2_ref[...]


def _tc_mlp(x, agg, W1, b1, W2, b2):
    nb = N // ROW_BLK
    return pl.pallas_call(
        _mlp_block,
        grid=(nb,),
        in_specs=[
            pl.BlockSpec((ROW_BLK, D), lambda i: (i, 0)),
            pl.BlockSpec((1, ROW_BLK, D), lambda i: (0, i, 0)),
            pl.BlockSpec((1, ROW_BLK, D), lambda i: (1, i, 0)),
            pl.BlockSpec((D, D), lambda i: (0, 0)),
            pl.BlockSpec((1, D), lambda i: (0, 0)),
            pl.BlockSpec((D, D), lambda i: (0, 0)),
            pl.BlockSpec((1, D), lambda i: (0, 0)),
        ],
        out_specs=pl.BlockSpec((ROW_BLK, D), lambda i: (i, 0)),
        out_shape=jax.ShapeDtypeStruct((N, D), jnp.float32),
    )(x, agg, agg, W1, b1.reshape(1, D), W2, b2.reshape(1, D))


def kernel(x, edge_index, W1, b1, W2, b2):
    rows = edge_index[:, 0]
    cols = edge_index[:, 1]
    pad = E_PAD - E
    rows_p = jnp.concatenate(
        [rows, jnp.full((pad,), N, dtype=jnp.int32)]
    ).reshape(NC, NS, K, CHUNK)
    cols_p = jnp.concatenate(
        [cols, jnp.zeros((pad,), dtype=jnp.int32)]
    ).reshape(NC, NS, K, CHUNK)
    agg = _sc_aggregate(x, rows_p, cols_p)
    return _tc_mlp(x, agg, W1, b1, W2, b2)


# speedup vs baseline: 1.2729x; 1.1402x over previous
import functools

import jax
import jax.numpy as jnp
from jax import lax
from jax.experimental import pallas as pl
from jax.experimental.pallas import tpu as pltpu
from jax.experimental.pallas import tpu_sc as plsc

N = 10000
D = 128
E = 320000

NC = 2
NS = 16
NW = NC * NS
CHUNK = 128
PER_W = -(-E // (NW * CHUNK)) * CHUNK
K = PER_W // CHUNK
E_PAD = PER_W * NW
STRIPE = 632
STRIPE_LAST = N - STRIPE * (NS - 1)
N_PAD = N + 8

ROW_BLK = 1000


def _sc_aggregate(x, rows_r, cols_r):
    mesh = plsc.VectorSubcoreMesh(
        core_axis_name="c", subcore_axis_name="s",
        num_cores=NC, num_subcores=NS)

    @functools.partial(
        pl.kernel,
        out_type=jax.ShapeDtypeStruct((NC, N, D), jnp.float32),
        mesh=mesh,
        scratch_types=[
            pltpu.VMEM((1, CHUNK), jnp.int32),
            pltpu.VMEM((1, CHUNK), jnp.int32),
            pltpu.VMEM((CHUNK, D), jnp.float32),
            pltpu.VMEM_SHARED((N_PAD, D), jnp.float32),
            pltpu.SemaphoreType.DMA,
        ],
    )
    def agg_kernel(x_hbm, rows_hbm, cols_hbm, out_hbm,
                   row_v, col_v, buf, agg_sh, sem):
        c = lax.axis_index("c")
        s = lax.axis_index("s")
        off = pl.multiple_of(s * STRIPE, 8)

        @pl.when(s < NS - 1)
        def _():
            pltpu.sync_copy(x_hbm.at[pl.ds(off, STRIPE)],
                            agg_sh.at[pl.ds(off, STRIPE)])

        @pl.when(s == NS - 1)
        def _():
            pltpu.sync_copy(x_hbm.at[pl.ds((NS - 1) * STRIPE, STRIPE_LAST)],
                            agg_sh.at[pl.ds((NS - 1) * STRIPE, STRIPE_LAST)])

        plsc.subcore_barrier()

        @pl.loop(0, K)
        def _(j):
            pltpu.sync_copy(rows_hbm.at[c, s, pl.ds(j, 1)], row_v)
            pltpu.sync_copy(cols_hbm.at[c, s, pl.ds(j, 1)], col_v)
            pltpu.async_copy(x_hbm.at[col_v.at[0]], buf, sem).wait()
            pltpu.sync_copy(buf, agg_sh.at[row_v.at[0]], add=True)

        plsc.subcore_barrier()

        @pl.when(s < NS - 1)
        def _():
            pltpu.sync_copy(agg_sh.at[pl.ds(off, STRIPE)],
                            out_hbm.at[c, pl.ds(off, STRIPE)])

        @pl.when(s == NS - 1)
        def _():
            pltpu.sync_copy(
                agg_sh.at[pl.ds((NS - 1) * STRIPE, STRIPE_LAST)],
                out_hbm.at[c, pl.ds((NS - 1) * STRIPE, STRIPE_LAST)])

    return agg_kernel(x, rows_r, cols_r)


def _mlp_block(x_ref, a0_ref, a1_ref, w1_ref, b1_ref, w2_ref, b2_ref, o_ref):
    a = a0_ref[0] + a1_ref[0] - x_ref[...]
    h = jnp.dot(a, w1_ref[...], preferred_element_type=jnp.float32,
                precision=lax.Precision.HIGHEST) + b1_ref[...]
    h = jnp.maximum(h, 0.0)
    o_ref[...] = jnp.dot(h, w2_ref[...], preferred_element_type=jnp.float32,
                         precision=lax.Precision.HIGHEST) + b2_ref[...]


def _tc_mlp(x, agg, W1, b1, W2, b2):
    nb = N // ROW_BLK
    return pl.pallas_call(
        _mlp_block,
        grid=(nb,),
        in_specs=[
            pl.BlockSpec((ROW_BLK, D), lambda i: (i, 0)),
            pl.BlockSpec((1, ROW_BLK, D), lambda i: (0, i, 0)),
            pl.BlockSpec((1, ROW_BLK, D), lambda i: (1, i, 0)),
            pl.BlockSpec((D, D), lambda i: (0, 0)),
            pl.BlockSpec((1, D), lambda i: (0, 0)),
            pl.BlockSpec((D, D), lambda i: (0, 0)),
            pl.BlockSpec((1, D), lambda i: (0, 0)),
        ],
        out_specs=pl.BlockSpec((ROW_BLK, D), lambda i: (i, 0)),
        out_shape=jax.ShapeDtypeStruct((N, D), jnp.float32),
    )(x, agg, agg, W1, b1.reshape(1, D), W2, b2.reshape(1, D))


def kernel(x, edge_index, W1, b1, W2, b2):
    rows = edge_index[:, 0]
    cols = edge_index[:, 1]
    pad = E_PAD - E
    rows_p = jnp.concatenate(
        [rows, jnp.full((pad,), N, dtype=jnp.int32)]
    ).reshape(NC, NS, K, CHUNK)
    cols_p = jnp.concatenate(
        [cols, jnp.zeros((pad,), dtype=jnp.int32)]
    ).reshape(NC, NS, K, CHUNK)
    agg = _sc_aggregate(x, rows_p, cols_p)
    return _tc_mlp(x, agg, W1, b1, W2, b2)


# speedup vs baseline: 1.4730x; 1.1572x over previous
import functools

import jax
import jax.numpy as jnp
from jax import lax
from jax.experimental import pallas as pl
from jax.experimental.pallas import tpu as pltpu
from jax.experimental.pallas import tpu_sc as plsc

N = 10000
D = 128
E = 320000

NC = 2
NS = 16
NW = NC * NS
CHUNK = 128
PER_W = -(-E // (NW * CHUNK)) * CHUNK
K = PER_W // CHUNK
E_PAD = PER_W * NW
STRIPE = 632
STRIPE_LAST = N - STRIPE * (NS - 1)
N_PAD = N + 8

ROW_BLK = 1000


def _sc_aggregate(x, rows_r, cols_r):
    mesh = plsc.VectorSubcoreMesh(
        core_axis_name="c", subcore_axis_name="s",
        num_cores=NC, num_subcores=NS)

    @functools.partial(
        pl.kernel,
        out_type=jax.ShapeDtypeStruct((NC, N, D), jnp.float32),
        mesh=mesh,
        scratch_types=[
            pltpu.VMEM((K, CHUNK), jnp.int32),
            pltpu.VMEM((K, CHUNK), jnp.int32),
            pltpu.VMEM((CHUNK, D), jnp.float32),
            pltpu.VMEM_SHARED((N_PAD, D), jnp.float32),
            pltpu.SemaphoreType.DMA,
        ],
    )
    def agg_kernel(x_hbm, rows_hbm, cols_hbm, out_hbm,
                   row_v, col_v, buf, agg_sh, sem):
        c = lax.axis_index("c")
        s = lax.axis_index("s")
        pltpu.sync_copy(rows_hbm.at[c, s], row_v)
        pltpu.sync_copy(cols_hbm.at[c, s], col_v)
        off = pl.multiple_of(s * STRIPE, 8)

        @pl.when(s < NS - 1)
        def _():
            pltpu.sync_copy(x_hbm.at[pl.ds(off, STRIPE)],
                            agg_sh.at[pl.ds(off, STRIPE)])

        @pl.when(s == NS - 1)
        def _():
            pltpu.sync_copy(x_hbm.at[pl.ds((NS - 1) * STRIPE, STRIPE_LAST)],
                            agg_sh.at[pl.ds((NS - 1) * STRIPE, STRIPE_LAST)])

        plsc.subcore_barrier()

        @pl.loop(0, K)
        def _(j):
            pltpu.async_copy(x_hbm.at[col_v.at[j]], buf, sem).wait()
            pltpu.sync_copy(buf, agg_sh.at[row_v.at[j]], add=True)

        plsc.subcore_barrier()

        @pl.when(s < NS - 1)
        def _():
            pltpu.sync_copy(agg_sh.at[pl.ds(off, STRIPE)],
                            out_hbm.at[c, pl.ds(off, STRIPE)])

        @pl.when(s == NS - 1)
        def _():
            pltpu.sync_copy(
                agg_sh.at[pl.ds((NS - 1) * STRIPE, STRIPE_LAST)],
                out_hbm.at[c, pl.ds((NS - 1) * STRIPE, STRIPE_LAST)])

    return agg_kernel(x, rows_r, cols_r)


def _mlp_block(x_ref, a0_ref, a1_ref, w1_ref, b1_ref, w2_ref, b2_ref, o_ref):
    a = a0_ref[0] + a1_ref[0] - x_ref[...]
    h = jnp.dot(a, w1_ref[...], preferred_element_type=jnp.float32,
                precision=lax.Precision.HIGHEST) + b1_ref[...]
    h = jnp.maximum(h, 0.0)
    o_ref[...] = jnp.dot(h, w2_ref[...], preferred_element_type=jnp.float32,
                         precision=lax.Precision.HIGHEST) + b2_ref[...]


def _tc_mlp(x, agg, W1, b1, W2, b2):
    nb = N // ROW_BLK
    return pl.pallas_call(
        _mlp_block,
        grid=(nb,),
        in_specs=[
            pl.BlockSpec((ROW_BLK, D), lambda i: (i, 0)),
            pl.BlockSpec((1, ROW_BLK, D), lambda i: (0, i, 0)),
            pl.BlockSpec((1, ROW_BLK, D), lambda i: (1, i, 0)),
            pl.BlockSpec((D, D), lambda i: (0, 0)),
            pl.BlockSpec((1, D), lambda i: (0, 0)),
            pl.BlockSpec((D, D), lambda i: (0, 0)),
            pl.BlockSpec((1, D), lambda i: (0, 0)),
        ],
        out_specs=pl.BlockSpec((ROW_BLK, D), lambda i: (i, 0)),
        out_shape=jax.ShapeDtypeStruct((N, D), jnp.float32),
    )(x, agg, agg, W1, b1.reshape(1, D), W2, b2.reshape(1, D))


def kernel(x, edge_index, W1, b1, W2, b2):
    rows = edge_index[:, 0]
    cols = edge_index[:, 1]
    pad = E_PAD - E
    rows_p = jnp.concatenate(
        [rows, jnp.full((pad,), N, dtype=jnp.int32)]
    ).reshape(NC, NS, K, CHUNK)
    cols_p = jnp.concatenate(
        [cols, jnp.zeros((pad,), dtype=jnp.int32)]
    ).reshape(NC, NS, K, CHUNK)
    agg = _sc_aggregate(x, rows_p, cols_p)
    return _tc_mlp(x, agg, W1, b1, W2, b2)
